# Initial kernel scaffold; baseline (speedup 1.0000x reference)
import jax
import jax.numpy as jnp
from jax.experimental import pallas as pl


def kernel(x, edge_index, mlp0_W1, mlp0_b1, mlp0_W2, mlp0_b2, bn0_gamma, bn0_beta, mlp1_W1, mlp1_b1, mlp1_W2, mlp1_b2, bn1_gamma, bn1_beta, gru_W_ih, gru_W_hh, gru_b_ih, gru_b_hh, last_W1, last_b1, last_W2, last_b2):
    raise NotImplementedError("write your pallas kernel here")


# speedup vs baseline: 1.5857x; 1.5857x over previous
import functools

import jax
import jax.numpy as jnp
from jax import lax
from jax.experimental import pallas as pl
from jax.experimental.pallas import tpu as pltpu
from jax.experimental.pallas import tpu_sc as plsc

N = 10000
D = 128
E = 160000

NC = 2
NS = 16
NW = NC * NS
CH = 128
EPW = 5120
E_PAD = NW * EPW
N_PAD = 10112
RPT = N_PAD // NS
TRASH = N


def _agg_body(x_hbm, zeros_hbm, ga_hbm, gb_hbm, da_hbm, db_hbm, out_hbm,
              gidx, didx, rows, agg_sh, sem):
    cid = lax.axis_index("c")
    sid = lax.axis_index("s")
    wid = sid * NC + cid
    base = wid * EPW
    r0 = sid * RPT

    pltpu.sync_copy(zeros_hbm.at[pl.ds(r0, RPT)], agg_sh.at[pl.ds(r0, RPT)])
    plsc.subcore_barrier()

    def chunk(c, carry):
        off = base + c * CH
        pltpu.sync_copy(ga_hbm.at[pl.ds(off, CH)], gidx)
        pltpu.sync_copy(da_hbm.at[pl.ds(off, CH)], didx)
        pltpu.async_copy(x_hbm.at[gidx], rows, sem).wait()
        pltpu.sync_copy(rows, agg_sh.at[didx], add=True)
        pltpu.sync_copy(gb_hbm.at[pl.ds(off, CH)], gidx)
        pltpu.sync_copy(db_hbm.at[pl.ds(off, CH)], didx)
        pltpu.async_copy(x_hbm.at[gidx], rows, sem).wait()
        pltpu.sync_copy(rows, agg_sh.at[didx], add=True)
        return carry

    lax.fori_loop(0, EPW // CH, chunk, 0)
    plsc.subcore_barrier()
    pltpu.sync_copy(agg_sh.at[pl.ds(r0, RPT)], out_hbm.at[cid, pl.ds(r0, RPT)])


@functools.cache
def _get_agg_call():
    return functools.partial(
        pl.kernel,
        out_type=jax.ShapeDtypeStruct((NC, N_PAD, D), jnp.float32),
        mesh=plsc.VectorSubcoreMesh(core_axis_name="c", subcore_axis_name="s"),
        scratch_types=[
            pltpu.VMEM((CH,), jnp.int32),
            pltpu.VMEM((CH,), jnp.int32),
            pltpu.VMEM((CH, D), jnp.float32),
            pltpu.VMEM_SHARED((N_PAD, D), jnp.float32),
            pltpu.SemaphoreType.DMA,
        ],
    )(_agg_body)


def _agg(x, zeros_rows, ga, gb, da, db):
    return _get_agg_call()(x, zeros_rows, ga, gb, da, db)


_B = 1000
_G = N // _B


def _mm(a, w):
    return jnp.dot(a.astype(jnp.bfloat16), w.astype(jnp.bfloat16),
                   preferred_element_type=jnp.float32)


def _layer0_body(x_ref, ag_ref, w1, b1, w2, b2, bns, bnb, wih, bih, bhh,
                 o_ref):
    t = x_ref[...] + ag_ref[0] + ag_ref[1]
    h = jnp.maximum(_mm(t, w1[...]) + b1[...], 0.0)
    xc = _mm(h, w2[...]) + b2[...]
    xc = xc * bns[...] + bnb[...]
    gi = _mm(xc, wih[...]) + bih[...]
    gh = bhh[...]
    r = jax.nn.sigmoid(gi[:, :D] + gh[:, :D])
    z = jax.nn.sigmoid(gi[:, D:2 * D] + gh[:, D:2 * D])
    n = jnp.tanh(gi[:, 2 * D:] + r * gh[:, 2 * D:])
    o_ref[...] = (1.0 - z) * n


def _layer1_body(x_ref, ag_ref, w1, b1, w2, b2, bns, bnb, wih, whh, bih, bhh,
                 lw1, lb1, lw2, lb2, o_ref):
    xb = x_ref[...]
    t = xb + ag_ref[0] + ag_ref[1]
    h = jnp.maximum(_mm(t, w1[...]) + b1[...], 0.0)
    xc = _mm(h, w2[...]) + b2[...]
    xc = xc * bns[...] + bnb[...]
    gi = _mm(xc, wih[...]) + bih[...]
    gh = _mm(xb, whh[...]) + bhh[...]
    r = jax.nn.sigmoid(gi[:, :D] + gh[:, :D])
    z = jax.nn.sigmoid(gi[:, D:2 * D] + gh[:, D:2 * D])
    n = jnp.tanh(gi[:, 2 * D:] + r * gh[:, 2 * D:])
    hn = (1.0 - z) * n + z * xb
    u = jnp.maximum(_mm(hn, lw1[...]) + lb1[...], 0.0)
    o_ref[...] = _mm(u, lw2[...]) + lb2[...]


def _w_spec(shape):
    return pl.BlockSpec(shape, lambda i: (0,) * len(shape))


_x_spec = pl.BlockSpec((_B, D), lambda i: (i, 0))
_ag_spec = pl.BlockSpec((NC, _B, D), lambda i: (0, i, 0))

_layer0_call = pl.pallas_call(
    _layer0_body,
    grid=(_G,),
    in_specs=[
        _x_spec, _ag_spec,
        _w_spec((D, D)), _w_spec((1, D)), _w_spec((D, D)), _w_spec((1, D)),
        _w_spec((1, D)), _w_spec((1, D)),
        _w_spec((D, 3 * D)), _w_spec((1, 3 * D)), _w_spec((1, 3 * D)),
    ],
    out_specs=_x_spec,
    out_shape=jax.ShapeDtypeStruct((N, D), jnp.float32),
)

_layer1_call = pl.pallas_call(
    _layer1_body,
    grid=(_G,),
    in_specs=[
        _x_spec, _ag_spec,
        _w_spec((D, D)), _w_spec((1, D)), _w_spec((D, D)), _w_spec((1, D)),
        _w_spec((1, D)), _w_spec((1, D)),
        _w_spec((D, 3 * D)), _w_spec((D, 3 * D)),
        _w_spec((1, 3 * D)), _w_spec((1, 3 * D)),
        _w_spec((D, D)), _w_spec((1, D)), _w_spec((D, D)), _w_spec((1, D)),
    ],
    out_specs=_x_spec,
    out_shape=jax.ShapeDtypeStruct((N, D), jnp.float32),
)


def kernel(x, edge_index, mlp0_W1, mlp0_b1, mlp0_W2, mlp0_b2, bn0_gamma,
           bn0_beta, mlp1_W1, mlp1_b1, mlp1_W2, mlp1_b2, bn1_gamma, bn1_beta,
           gru_W_ih, gru_W_hh, gru_b_ih, gru_b_hh, last_W1, last_b1, last_W2,
           last_b2):
    src = edge_index[0]
    dst = edge_index[1]

    a = jnp.minimum(src, dst)
    b = jnp.maximum(src, dst)
    skey = jnp.sort(a * N + b)
    a_s = skey // N
    b_s = skey - a_s * N
    uniq = jnp.concatenate(
        [jnp.ones((1,), jnp.bool_), skey[1:] != skey[:-1]])
    da = jnp.where(uniq, a_s, TRASH)
    db = jnp.where(uniq & (a_s != b_s), b_s, TRASH)

    pad_i = jnp.zeros((E_PAD - E,), jnp.int32)
    pad_t = jnp.full((E_PAD - E,), TRASH, jnp.int32)
    ga = jnp.concatenate([b_s.astype(jnp.int32), pad_i])
    gb = jnp.concatenate([a_s.astype(jnp.int32), pad_i])
    da = jnp.concatenate([da.astype(jnp.int32), pad_t])
    db = jnp.concatenate([db.astype(jnp.int32), pad_t])
    zeros_rows = jnp.zeros((N_PAD, D), jnp.float32)

    wihT = gru_W_ih.T
    whhT = gru_W_hh.T
    bih = gru_b_ih.reshape(1, 3 * D)
    bhh = gru_b_hh.reshape(1, 3 * D)
    inv = 1.0 / jnp.sqrt(jnp.float32(1.0 + 1e-5))
    bns0 = (bn0_gamma * inv).reshape(1, D)
    bns1 = (bn1_gamma * inv).reshape(1, D)

    x_r = lax.reduce_precision(x, exponent_bits=8, mantissa_bits=7)
    aggp0 = _agg(x_r, zeros_rows, ga, gb, da, db)
    x1 = _layer0_call(
        x, aggp0, mlp0_W1, mlp0_b1.reshape(1, D), mlp0_W2,
        mlp0_b2.reshape(1, D), bns0, bn0_beta.reshape(1, D), wihT, bih, bhh)
    x1_r = lax.reduce_precision(x1, exponent_bits=8, mantissa_bits=7)
    aggp1 = _agg(x1_r, zeros_rows, ga, gb, da, db)
    out = _layer1_call(
        x1, aggp1, mlp1_W1, mlp1_b1.reshape(1, D), mlp1_W2,
        mlp1_b2.reshape(1, D), bns1, bn1_beta.reshape(1, D), wihT, whhT, bih,
        bhh, last_W1, last_b1.reshape(1, D), last_W2, last_b2.reshape(1, D))
    return out


# speedup vs baseline: 1.9125x; 1.2061x over previous
import functools

import jax
import jax.numpy as jnp
from jax import lax
from jax.experimental import pallas as pl
from jax.experimental.pallas import tpu as pltpu
from jax.experimental.pallas import tpu_sc as plsc

N = 10000
D = 128
E = 160000

NC = 2
NS = 16
NW = NC * NS
CH = 128
EPW = 5120
E_PAD = NW * EPW
N_PAD = 10112
RPT = N_PAD // NS
TRASH = N


NJ = 2 * (EPW // CH)
HALF = NJ // 2


def _agg_body(x_hbm, zeros_hbm, garr_hbm, darr_hbm, out_hbm,
              gidx_all, didx_all, rows0, rows1, agg_sh, sem0, sem1):
    cid = lax.axis_index("c")
    sid = lax.axis_index("s")
    wid = sid * NC + cid
    r0 = sid * RPT

    pltpu.sync_copy(zeros_hbm.at[pl.ds(r0, RPT)], agg_sh.at[pl.ds(r0, RPT)])
    plsc.subcore_barrier()

    def start(j, buf, sem):
        pltpu.async_copy(x_hbm.at[gidx_all.at[j]], buf, sem)

    def wait(buf, sem):
        pltpu.make_async_copy(x_hbm.at[pl.ds(0, CH)], buf, sem).wait()

    def scatter(j, buf):
        pltpu.sync_copy(buf, agg_sh.at[didx_all.at[j]], add=True)

    for p in range(2):
        pltpu.sync_copy(garr_hbm.at[pl.ds(wid * NJ + p * HALF, HALF)],
                        gidx_all)
        pltpu.sync_copy(darr_hbm.at[pl.ds(wid * NJ + p * HALF, HALF)],
                        didx_all)
        start(0, rows0, sem0)

        def body(k, carry):
            j0 = 2 * k
            j1 = j0 + 1
            wait(rows0, sem0)
            start(j1, rows1, sem1)
            scatter(j0, rows0)
            wait(rows1, sem1)

            @pl.when(k < HALF // 2 - 1)
            def _():
                start(j1 + 1, rows0, sem0)

            scatter(j1, rows1)
            return carry

        lax.fori_loop(0, HALF // 2, body, 0)
    plsc.subcore_barrier()
    pltpu.sync_copy(agg_sh.at[pl.ds(r0, RPT)], out_hbm.at[cid, pl.ds(r0, RPT)])


@functools.cache
def _get_agg_call():
    return functools.partial(
        pl.kernel,
        out_type=jax.ShapeDtypeStruct((NC, N_PAD, D), jnp.float32),
        mesh=plsc.VectorSubcoreMesh(core_axis_name="c", subcore_axis_name="s"),
        scratch_types=[
            pltpu.VMEM((HALF, CH), jnp.int32),
            pltpu.VMEM((HALF, CH), jnp.int32),
            pltpu.VMEM((CH, D), jnp.float32),
            pltpu.VMEM((CH, D), jnp.float32),
            pltpu.VMEM_SHARED((N_PAD, D), jnp.float32),
            pltpu.SemaphoreType.DMA,
            pltpu.SemaphoreType.DMA,
        ],
    )(_agg_body)


def _agg(x, zeros_rows, garr, darr):
    return _get_agg_call()(x, zeros_rows, garr, darr)


_B = 1000
_G = N // _B


def _mm(a, w):
    return jnp.dot(a.astype(jnp.bfloat16), w.astype(jnp.bfloat16),
                   preferred_element_type=jnp.float32)


def _layer0_body(x_ref, ag_ref, w1, b1, w2, b2, bns, bnb, wih, bih, bhh,
                 o_ref):
    t = x_ref[...] + ag_ref[0] + ag_ref[1]
    h = jnp.maximum(_mm(t, w1[...]) + b1[...], 0.0)
    xc = _mm(h, w2[...]) + b2[...]
    xc = xc * bns[...] + bnb[...]
    gi = _mm(xc, wih[...]) + bih[...]
    gh = bhh[...]
    r = jax.nn.sigmoid(gi[:, :D] + gh[:, :D])
    z = jax.nn.sigmoid(gi[:, D:2 * D] + gh[:, D:2 * D])
    n = jnp.tanh(gi[:, 2 * D:] + r * gh[:, 2 * D:])
    o_ref[...] = (1.0 - z) * n


def _layer1_body(x_ref, ag_ref, w1, b1, w2, b2, bns, bnb, wih, whh, bih, bhh,
                 lw1, lb1, lw2, lb2, o_ref):
    xb = x_ref[...]
    t = xb + ag_ref[0] + ag_ref[1]
    h = jnp.maximum(_mm(t, w1[...]) + b1[...], 0.0)
    xc = _mm(h, w2[...]) + b2[...]
    xc = xc * bns[...] + bnb[...]
    gi = _mm(xc, wih[...]) + bih[...]
    gh = _mm(xb, whh[...]) + bhh[...]
    r = jax.nn.sigmoid(gi[:, :D] + gh[:, :D])
    z = jax.nn.sigmoid(gi[:, D:2 * D] + gh[:, D:2 * D])
    n = jnp.tanh(gi[:, 2 * D:] + r * gh[:, 2 * D:])
    hn = (1.0 - z) * n + z * xb
    u = jnp.maximum(_mm(hn, lw1[...]) + lb1[...], 0.0)
    o_ref[...] = _mm(u, lw2[...]) + lb2[...]


def _w_spec(shape):
    return pl.BlockSpec(shape, lambda i: (0,) * len(shape))


_x_spec = pl.BlockSpec((_B, D), lambda i: (i, 0))
_ag_spec = pl.BlockSpec((NC, _B, D), lambda i: (0, i, 0))

_layer0_call = pl.pallas_call(
    _layer0_body,
    grid=(_G,),
    in_specs=[
        _x_spec, _ag_spec,
        _w_spec((D, D)), _w_spec((1, D)), _w_spec((D, D)), _w_spec((1, D)),
        _w_spec((1, D)), _w_spec((1, D)),
        _w_spec((D, 3 * D)), _w_spec((1, 3 * D)), _w_spec((1, 3 * D)),
    ],
    out_specs=_x_spec,
    out_shape=jax.ShapeDtypeStruct((N, D), jnp.float32),
)

_layer1_call = pl.pallas_call(
    _layer1_body,
    grid=(_G,),
    in_specs=[
        _x_spec, _ag_spec,
        _w_spec((D, D)), _w_spec((1, D)), _w_spec((D, D)), _w_spec((1, D)),
        _w_spec((1, D)), _w_spec((1, D)),
        _w_spec((D, 3 * D)), _w_spec((D, 3 * D)),
        _w_spec((1, 3 * D)), _w_spec((1, 3 * D)),
        _w_spec((D, D)), _w_spec((1, D)), _w_spec((D, D)), _w_spec((1, D)),
    ],
    out_specs=_x_spec,
    out_shape=jax.ShapeDtypeStruct((N, D), jnp.float32),
)


def kernel(x, edge_index, mlp0_W1, mlp0_b1, mlp0_W2, mlp0_b2, bn0_gamma,
           bn0_beta, mlp1_W1, mlp1_b1, mlp1_W2, mlp1_b2, bn1_gamma, bn1_beta,
           gru_W_ih, gru_W_hh, gru_b_ih, gru_b_hh, last_W1, last_b1, last_W2,
           last_b2):
    src = edge_index[0]
    dst = edge_index[1]

    a = jnp.minimum(src, dst)
    b = jnp.maximum(src, dst)
    skey = jnp.sort(a * N + b)
    a_s = skey // N
    b_s = skey - a_s * N
    uniq = jnp.concatenate(
        [jnp.ones((1,), jnp.bool_), skey[1:] != skey[:-1]])
    da = jnp.where(uniq, a_s, TRASH)
    db = jnp.where(uniq & (a_s != b_s), b_s, TRASH)

    pad_i = jnp.zeros((E_PAD - E,), jnp.int32)
    pad_t = jnp.full((E_PAD - E,), TRASH, jnp.int32)
    ga = jnp.concatenate([b_s.astype(jnp.int32), pad_i])
    gb = jnp.concatenate([a_s.astype(jnp.int32), pad_i])
    da = jnp.concatenate([da.astype(jnp.int32), pad_t])
    db = jnp.concatenate([db.astype(jnp.int32), pad_t])
    half = NJ // 2
    garr = jnp.concatenate([ga.reshape(NW, half, CH), gb.reshape(NW, half, CH)],
                           axis=1).reshape(NW * NJ, CH)
    darr = jnp.concatenate([da.reshape(NW, half, CH), db.reshape(NW, half, CH)],
                           axis=1).reshape(NW * NJ, CH)
    zeros_rows = jnp.zeros((N_PAD, D), jnp.float32)

    wihT = gru_W_ih.T
    whhT = gru_W_hh.T
    bih = gru_b_ih.reshape(1, 3 * D)
    bhh = gru_b_hh.reshape(1, 3 * D)
    inv = 1.0 / jnp.sqrt(jnp.float32(1.0 + 1e-5))
    bns0 = (bn0_gamma * inv).reshape(1, D)
    bns1 = (bn1_gamma * inv).reshape(1, D)

    x_r = lax.reduce_precision(x, exponent_bits=8, mantissa_bits=7)
    aggp0 = _agg(x_r, zeros_rows, garr, darr)
    x1 = _layer0_call(
        x, aggp0, mlp0_W1, mlp0_b1.reshape(1, D), mlp0_W2,
        mlp0_b2.reshape(1, D), bns0, bn0_beta.reshape(1, D), wihT, bih, bhh)
    x1_r = lax.reduce_precision(x1, exponent_bits=8, mantissa_bits=7)
    aggp1 = _agg(x1_r, zeros_rows, garr, darr)
    out = _layer1_call(
        x1, aggp1, mlp1_W1, mlp1_b1.reshape(1, D), mlp1_W2,
        mlp1_b2.reshape(1, D), bns1, bn1_beta.reshape(1, D), wihT, whhT, bih,
        bhh, last_W1, last_b1.reshape(1, D), last_W2, last_b2.reshape(1, D))
    return out


# speedup vs baseline: 2.2801x; 1.1922x over previous
import functools

import jax
import jax.numpy as jnp
from jax import lax
from jax.experimental import pallas as pl
from jax.experimental.pallas import tpu as pltpu
from jax.experimental.pallas import tpu_sc as plsc

N = 10000
D = 128
E = 160000

NC = 2
NS = 16
NW = NC * NS
CH = 128
EPW = 5120
E_PAD = NW * EPW
N_PAD = 10112
RPT = N_PAD // NS
TRASH = N


CA = 48
CB = 32
ROWS_A = 2 * CA
ROWS_B = 2 * CB
SPLIT_ROW = NS * ROWS_A


def _agg_body(x_hbm, zeros_hbm, garr_hbm, darr_hbm, out_hbm,
              gidx_all, didx_all, rows0, rows1, agg_sh, sem0, sem1):
    cid = lax.axis_index("c")
    sid = lax.axis_index("s")
    r0 = sid * RPT
    is0 = cid == 0
    base = jnp.where(is0, sid * ROWS_A, SPLIT_ROW + sid * ROWS_B)
    n_chunks = jnp.where(is0, CA, CB)

    pltpu.sync_copy(zeros_hbm.at[pl.ds(r0, RPT)], agg_sh.at[pl.ds(r0, RPT)])
    plsc.subcore_barrier()

    def start(j, buf, sem):
        pltpu.async_copy(x_hbm.at[gidx_all.at[j]], buf, sem)

    def wait(buf, sem):
        pltpu.make_async_copy(x_hbm.at[pl.ds(0, CH)], buf, sem).wait()

    def scatter(j, buf):
        pltpu.sync_copy(buf, agg_sh.at[didx_all.at[j]], add=True)

    for p in range(2):
        poff = base + p * n_chunks

        @pl.when(is0)
        def _():
            pltpu.sync_copy(garr_hbm.at[pl.ds(poff, CA)],
                            gidx_all.at[pl.ds(0, CA)])
            pltpu.sync_copy(darr_hbm.at[pl.ds(poff, CA)],
                            didx_all.at[pl.ds(0, CA)])

        @pl.when(jnp.logical_not(is0))
        def _():
            pltpu.sync_copy(garr_hbm.at[pl.ds(poff, CB)],
                            gidx_all.at[pl.ds(0, CB)])
            pltpu.sync_copy(darr_hbm.at[pl.ds(poff, CB)],
                            didx_all.at[pl.ds(0, CB)])

        start(0, rows0, sem0)
        n_pairs = n_chunks // 2

        def body(k, carry):
            j0 = 2 * k
            j1 = j0 + 1
            wait(rows0, sem0)
            start(j1, rows1, sem1)
            scatter(j0, rows0)
            wait(rows1, sem1)

            @pl.when(k < n_pairs - 1)
            def _():
                start(j1 + 1, rows0, sem0)

            scatter(j1, rows1)
            return carry

        lax.fori_loop(0, n_pairs, body, 0)
    plsc.subcore_barrier()
    pltpu.sync_copy(agg_sh.at[pl.ds(r0, RPT)], out_hbm.at[cid, pl.ds(r0, RPT)])


@functools.cache
def _get_agg_call():
    return functools.partial(
        pl.kernel,
        out_type=jax.ShapeDtypeStruct((NC, N_PAD, D), jnp.float32),
        mesh=plsc.VectorSubcoreMesh(core_axis_name="c", subcore_axis_name="s"),
        scratch_types=[
            pltpu.VMEM((CA, CH), jnp.int32),
            pltpu.VMEM((CA, CH), jnp.int32),
            pltpu.VMEM((CH, D), jnp.float32),
            pltpu.VMEM((CH, D), jnp.float32),
            pltpu.VMEM_SHARED((N_PAD, D), jnp.float32),
            pltpu.SemaphoreType.DMA,
            pltpu.SemaphoreType.DMA,
        ],
    )(_agg_body)


def _agg(x, zeros_rows, garr, darr):
    return _get_agg_call()(x, zeros_rows, garr, darr)


_B = 1000
_G = N // _B


def _mm(a, w):
    return jnp.dot(a.astype(jnp.bfloat16), w.astype(jnp.bfloat16),
                   preferred_element_type=jnp.float32)


def _layer0_body(x_ref, ag_ref, w1, b1, w2, b2, bns, bnb, wih, bih, bhh,
                 o_ref):
    t = x_ref[...] + ag_ref[0] + ag_ref[1]
    h = jnp.maximum(_mm(t, w1[...]) + b1[...], 0.0)
    xc = _mm(h, w2[...]) + b2[...]
    xc = xc * bns[...] + bnb[...]
    gi = _mm(xc, wih[...]) + bih[...]
    gh = bhh[...]
    r = jax.nn.sigmoid(gi[:, :D] + gh[:, :D])
    z = jax.nn.sigmoid(gi[:, D:2 * D] + gh[:, D:2 * D])
    n = jnp.tanh(gi[:, 2 * D:] + r * gh[:, 2 * D:])
    o_ref[...] = (1.0 - z) * n


def _layer1_body(x_ref, ag_ref, w1, b1, w2, b2, bns, bnb, wih, whh, bih, bhh,
                 lw1, lb1, lw2, lb2, o_ref):
    xb = x_ref[...]
    t = xb + ag_ref[0] + ag_ref[1]
    h = jnp.maximum(_mm(t, w1[...]) + b1[...], 0.0)
    xc = _mm(h, w2[...]) + b2[...]
    xc = xc * bns[...] + bnb[...]
    gi = _mm(xc, wih[...]) + bih[...]
    gh = _mm(xb, whh[...]) + bhh[...]
    r = jax.nn.sigmoid(gi[:, :D] + gh[:, :D])
    z = jax.nn.sigmoid(gi[:, D:2 * D] + gh[:, D:2 * D])
    n = jnp.tanh(gi[:, 2 * D:] + r * gh[:, 2 * D:])
    hn = (1.0 - z) * n + z * xb
    u = jnp.maximum(_mm(hn, lw1[...]) + lb1[...], 0.0)
    o_ref[...] = _mm(u, lw2[...]) + lb2[...]


def _w_spec(shape):
    return pl.BlockSpec(shape, lambda i: (0,) * len(shape))


_x_spec = pl.BlockSpec((_B, D), lambda i: (i, 0))
_ag_spec = pl.BlockSpec((NC, _B, D), lambda i: (0, i, 0))

_layer0_call = pl.pallas_call(
    _layer0_body,
    grid=(_G,),
    in_specs=[
        _x_spec, _ag_spec,
        _w_spec((D, D)), _w_spec((1, D)), _w_spec((D, D)), _w_spec((1, D)),
        _w_spec((1, D)), _w_spec((1, D)),
        _w_spec((D, 3 * D)), _w_spec((1, 3 * D)), _w_spec((1, 3 * D)),
    ],
    out_specs=_x_spec,
    out_shape=jax.ShapeDtypeStruct((N, D), jnp.float32),
)

_layer1_call = pl.pallas_call(
    _layer1_body,
    grid=(_G,),
    in_specs=[
        _x_spec, _ag_spec,
        _w_spec((D, D)), _w_spec((1, D)), _w_spec((D, D)), _w_spec((1, D)),
        _w_spec((1, D)), _w_spec((1, D)),
        _w_spec((D, 3 * D)), _w_spec((D, 3 * D)),
        _w_spec((1, 3 * D)), _w_spec((1, 3 * D)),
        _w_spec((D, D)), _w_spec((1, D)), _w_spec((D, D)), _w_spec((1, D)),
    ],
    out_specs=_x_spec,
    out_shape=jax.ShapeDtypeStruct((N, D), jnp.float32),
)


def kernel(x, edge_index, mlp0_W1, mlp0_b1, mlp0_W2, mlp0_b2, bn0_gamma,
           bn0_beta, mlp1_W1, mlp1_b1, mlp1_W2, mlp1_b2, bn1_gamma, bn1_beta,
           gru_W_ih, gru_W_hh, gru_b_ih, gru_b_hh, last_W1, last_b1, last_W2,
           last_b2):
    src = edge_index[0]
    dst = edge_index[1]

    a = jnp.minimum(src, dst)
    b = jnp.maximum(src, dst)
    skey = lax.sort(a * N + b, is_stable=False)
    a_s = skey // N
    b_s = skey - a_s * N
    uniq = jnp.concatenate(
        [jnp.ones((1,), jnp.bool_), skey[1:] != skey[:-1]])
    da = jnp.where(uniq, a_s, TRASH)
    db = jnp.where(uniq & (a_s != b_s), b_s, TRASH)

    pad_i = jnp.zeros((E_PAD - E,), jnp.int32)
    pad_t = jnp.full((E_PAD - E,), TRASH, jnp.int32)
    ga = jnp.concatenate([b_s.astype(jnp.int32), pad_i])
    gb = jnp.concatenate([a_s.astype(jnp.int32), pad_i])
    da = jnp.concatenate([da.astype(jnp.int32), pad_t])
    db = jnp.concatenate([db.astype(jnp.int32), pad_t])
    S = NS * CA * CH
    garr = jnp.concatenate([
        jnp.concatenate([ga[:S].reshape(NS, CA, CH),
                         gb[:S].reshape(NS, CA, CH)], axis=1).reshape(-1, CH),
        jnp.concatenate([ga[S:].reshape(NS, CB, CH),
                         gb[S:].reshape(NS, CB, CH)], axis=1).reshape(-1, CH),
    ])
    darr = jnp.concatenate([
        jnp.concatenate([da[:S].reshape(NS, CA, CH),
                         db[:S].reshape(NS, CA, CH)], axis=1).reshape(-1, CH),
        jnp.concatenate([da[S:].reshape(NS, CB, CH),
                         db[S:].reshape(NS, CB, CH)], axis=1).reshape(-1, CH),
    ])
    zeros_rows = jnp.zeros((N_PAD, D), jnp.float32)

    wihT = gru_W_ih.T
    whhT = gru_W_hh.T
    bih = gru_b_ih.reshape(1, 3 * D)
    bhh = gru_b_hh.reshape(1, 3 * D)
    inv = 1.0 / jnp.sqrt(jnp.float32(1.0 + 1e-5))
    bns0 = (bn0_gamma * inv).reshape(1, D)
    bns1 = (bn1_gamma * inv).reshape(1, D)

    x_r = lax.reduce_precision(x, exponent_bits=8, mantissa_bits=7)
    aggp0 = _agg(x_r, zeros_rows, garr, darr)
    x1 = _layer0_call(
        x, aggp0, mlp0_W1, mlp0_b1.reshape(1, D), mlp0_W2,
        mlp0_b2.reshape(1, D), bns0, bn0_beta.reshape(1, D), wihT, bih, bhh)
    x1_r = lax.reduce_precision(x1, exponent_bits=8, mantissa_bits=7)
    aggp1 = _agg(x1_r, zeros_rows, garr, darr)
    out = _layer1_call(
        x1, aggp1, mlp1_W1, mlp1_b1.reshape(1, D), mlp1_W2,
        mlp1_b2.reshape(1, D), bns1, bn1_beta.reshape(1, D), wihT, whhT, bih,
        bhh, last_W1, last_b1.reshape(1, D), last_W2, last_b2.reshape(1, D))
    return out


# speedup vs baseline: 2.2834x; 1.0014x over previous
import functools

import jax
import jax.numpy as jnp
from jax import lax
from jax.experimental import pallas as pl
from jax.experimental.pallas import tpu as pltpu
from jax.experimental.pallas import tpu_sc as plsc

N = 10000
D = 128
E = 160000

NC = 2
NS = 16
NW = NC * NS
CH = 128
EPW = 5120
E_PAD = NW * EPW
N_PAD = 10112
RPT = N_PAD // NS
TRASH = N


CA = 56
CB = 24
ROWS_A = 2 * CA
ROWS_B = 2 * CB
SPLIT_ROW = NS * ROWS_A


def _agg_body(x_hbm, zeros_hbm, garr_hbm, darr_hbm, out_hbm,
              gidx_all, didx_all, rows0, rows1, agg_sh, sem0, sem1):
    cid = lax.axis_index("c")
    sid = lax.axis_index("s")
    r0 = sid * RPT
    is0 = cid == 0
    base = jnp.where(is0, sid * ROWS_A, SPLIT_ROW + sid * ROWS_B)
    n_chunks = jnp.where(is0, CA, CB)

    pltpu.sync_copy(zeros_hbm.at[pl.ds(r0, RPT)], agg_sh.at[pl.ds(r0, RPT)])
    plsc.subcore_barrier()

    def start(j, buf, sem):
        pltpu.async_copy(x_hbm.at[gidx_all.at[j]], buf, sem)

    def wait(buf, sem):
        pltpu.make_async_copy(x_hbm.at[pl.ds(0, CH)], buf, sem).wait()

    def scatter(j, buf):
        pltpu.sync_copy(buf, agg_sh.at[didx_all.at[j]], add=True)

    for p in range(2):
        poff = base + p * n_chunks

        @pl.when(is0)
        def _():
            pltpu.sync_copy(garr_hbm.at[pl.ds(poff, CA)],
                            gidx_all.at[pl.ds(0, CA)])
            pltpu.sync_copy(darr_hbm.at[pl.ds(poff, CA)],
                            didx_all.at[pl.ds(0, CA)])

        @pl.when(jnp.logical_not(is0))
        def _():
            pltpu.sync_copy(garr_hbm.at[pl.ds(poff, CB)],
                            gidx_all.at[pl.ds(0, CB)])
            pltpu.sync_copy(darr_hbm.at[pl.ds(poff, CB)],
                            didx_all.at[pl.ds(0, CB)])

        start(0, rows0, sem0)
        n_pairs = n_chunks // 2

        def body(k, carry):
            j0 = 2 * k
            j1 = j0 + 1
            wait(rows0, sem0)
            start(j1, rows1, sem1)
            scatter(j0, rows0)
            wait(rows1, sem1)

            @pl.when(k < n_pairs - 1)
            def _():
                start(j1 + 1, rows0, sem0)

            scatter(j1, rows1)
            return carry

        lax.fori_loop(0, n_pairs, body, 0)
    plsc.subcore_barrier()
    pltpu.sync_copy(agg_sh.at[pl.ds(r0, RPT)], out_hbm.at[cid, pl.ds(r0, RPT)])


@functools.cache
def _get_agg_call():
    return functools.partial(
        pl.kernel,
        out_type=jax.ShapeDtypeStruct((NC, N_PAD, D), jnp.float32),
        mesh=plsc.VectorSubcoreMesh(core_axis_name="c", subcore_axis_name="s"),
        scratch_types=[
            pltpu.VMEM((CA, CH), jnp.int32),
            pltpu.VMEM((CA, CH), jnp.int32),
            pltpu.VMEM((CH, D), jnp.float32),
            pltpu.VMEM((CH, D), jnp.float32),
            pltpu.VMEM_SHARED((N_PAD, D), jnp.float32),
            pltpu.SemaphoreType.DMA,
            pltpu.SemaphoreType.DMA,
        ],
    )(_agg_body)


def _agg(x, zeros_rows, garr, darr):
    return _get_agg_call()(x, zeros_rows, garr, darr)


_B = 1000
_G = N // _B


def _mm(a, w):
    return jnp.dot(a.astype(jnp.bfloat16), w.astype(jnp.bfloat16),
                   preferred_element_type=jnp.float32)


def _layer0_body(x_ref, ag_ref, w1, b1, w2, b2, bns, bnb, wih, bih, bhh,
                 o_ref):
    t = x_ref[...] + ag_ref[0] + ag_ref[1]
    h = jnp.maximum(_mm(t, w1[...]) + b1[...], 0.0)
    xc = _mm(h, w2[...]) + b2[...]
    xc = xc * bns[...] + bnb[...]
    gi = _mm(xc, wih[...]) + bih[...]
    gh = bhh[...]
    r = jax.nn.sigmoid(gi[:, :D] + gh[:, :D])
    z = jax.nn.sigmoid(gi[:, D:2 * D] + gh[:, D:2 * D])
    n = jnp.tanh(gi[:, 2 * D:] + r * gh[:, 2 * D:])
    o_ref[...] = (1.0 - z) * n


def _layer1_body(x_ref, ag_ref, w1, b1, w2, b2, bns, bnb, wih, whh, bih, bhh,
                 lw1, lb1, lw2, lb2, o_ref):
    xb = x_ref[...]
    t = xb + ag_ref[0] + ag_ref[1]
    h = jnp.maximum(_mm(t, w1[...]) + b1[...], 0.0)
    xc = _mm(h, w2[...]) + b2[...]
    xc = xc * bns[...] + bnb[...]
    gi = _mm(xc, wih[...]) + bih[...]
    gh = _mm(xb, whh[...]) + bhh[...]
    r = jax.nn.sigmoid(gi[:, :D] + gh[:, :D])
    z = jax.nn.sigmoid(gi[:, D:2 * D] + gh[:, D:2 * D])
    n = jnp.tanh(gi[:, 2 * D:] + r * gh[:, 2 * D:])
    hn = (1.0 - z) * n + z * xb
    u = jnp.maximum(_mm(hn, lw1[...]) + lb1[...], 0.0)
    o_ref[...] = _mm(u, lw2[...]) + lb2[...]


def _w_spec(shape):
    return pl.BlockSpec(shape, lambda i: (0,) * len(shape))


_x_spec = pl.BlockSpec((_B, D), lambda i: (i, 0))
_ag_spec = pl.BlockSpec((NC, _B, D), lambda i: (0, i, 0))

_layer0_call = pl.pallas_call(
    _layer0_body,
    grid=(_G,),
    in_specs=[
        _x_spec, _ag_spec,
        _w_spec((D, D)), _w_spec((1, D)), _w_spec((D, D)), _w_spec((1, D)),
        _w_spec((1, D)), _w_spec((1, D)),
        _w_spec((D, 3 * D)), _w_spec((1, 3 * D)), _w_spec((1, 3 * D)),
    ],
    out_specs=_x_spec,
    out_shape=jax.ShapeDtypeStruct((N, D), jnp.float32),
)

_layer1_call = pl.pallas_call(
    _layer1_body,
    grid=(_G,),
    in_specs=[
        _x_spec, _ag_spec,
        _w_spec((D, D)), _w_spec((1, D)), _w_spec((D, D)), _w_spec((1, D)),
        _w_spec((1, D)), _w_spec((1, D)),
        _w_spec((D, 3 * D)), _w_spec((D, 3 * D)),
        _w_spec((1, 3 * D)), _w_spec((1, 3 * D)),
        _w_spec((D, D)), _w_spec((1, D)), _w_spec((D, D)), _w_spec((1, D)),
    ],
    out_specs=_x_spec,
    out_shape=jax.ShapeDtypeStruct((N, D), jnp.float32),
)


def kernel(x, edge_index, mlp0_W1, mlp0_b1, mlp0_W2, mlp0_b2, bn0_gamma,
           bn0_beta, mlp1_W1, mlp1_b1, mlp1_W2, mlp1_b2, bn1_gamma, bn1_beta,
           gru_W_ih, gru_W_hh, gru_b_ih, gru_b_hh, last_W1, last_b1, last_W2,
           last_b2):
    src = edge_index[0]
    dst = edge_index[1]

    a = jnp.minimum(src, dst)
    b = jnp.maximum(src, dst)
    skey = lax.sort(a * N + b, is_stable=False)
    a_s = skey // N
    b_s = skey - a_s * N
    uniq = jnp.concatenate(
        [jnp.ones((1,), jnp.bool_), skey[1:] != skey[:-1]])
    da = jnp.where(uniq, a_s, TRASH)
    db = jnp.where(uniq & (a_s != b_s), b_s, TRASH)

    pad_i = jnp.zeros((E_PAD - E,), jnp.int32)
    pad_t = jnp.full((E_PAD - E,), TRASH, jnp.int32)
    ga = jnp.concatenate([b_s.astype(jnp.int32), pad_i])
    gb = jnp.concatenate([a_s.astype(jnp.int32), pad_i])
    da = jnp.concatenate([da.astype(jnp.int32), pad_t])
    db = jnp.concatenate([db.astype(jnp.int32), pad_t])
    S = NS * CA * CH
    garr = jnp.concatenate([
        jnp.concatenate([ga[:S].reshape(NS, CA, CH),
                         gb[:S].reshape(NS, CA, CH)], axis=1).reshape(-1, CH),
        jnp.concatenate([ga[S:].reshape(NS, CB, CH),
                         gb[S:].reshape(NS, CB, CH)], axis=1).reshape(-1, CH),
    ])
    darr = jnp.concatenate([
        jnp.concatenate([da[:S].reshape(NS, CA, CH),
                         db[:S].reshape(NS, CA, CH)], axis=1).reshape(-1, CH),
        jnp.concatenate([da[S:].reshape(NS, CB, CH),
                         db[S:].reshape(NS, CB, CH)], axis=1).reshape(-1, CH),
    ])
    zeros_rows = jnp.zeros((N_PAD, D), jnp.float32)

    wihT = gru_W_ih.T
    whhT = gru_W_hh.T
    bih = gru_b_ih.reshape(1, 3 * D)
    bhh = gru_b_hh.reshape(1, 3 * D)
    inv = 1.0 / jnp.sqrt(jnp.float32(1.0 + 1e-5))
    bns0 = (bn0_gamma * inv).reshape(1, D)
    bns1 = (bn1_gamma * inv).reshape(1, D)

    x_r = lax.reduce_precision(x, exponent_bits=8, mantissa_bits=7)
    aggp0 = _agg(x_r, zeros_rows, garr, darr)
    x1 = _layer0_call(
        x, aggp0, mlp0_W1, mlp0_b1.reshape(1, D), mlp0_W2,
        mlp0_b2.reshape(1, D), bns0, bn0_beta.reshape(1, D), wihT, bih, bhh)
    x1_r = lax.reduce_precision(x1, exponent_bits=8, mantissa_bits=7)
    aggp1 = _agg(x1_r, zeros_rows, garr, darr)
    out = _layer1_call(
        x1, aggp1, mlp1_W1, mlp1_b1.reshape(1, D), mlp1_W2,
        mlp1_b2.reshape(1, D), bns1, bn1_beta.reshape(1, D), wihT, whhT, bih,
        bhh, last_W1, last_b1.reshape(1, D), last_W2, last_b2.reshape(1, D))
    return out


# speedup vs baseline: 2.3984x; 1.0504x over previous
import functools

import jax
import jax.numpy as jnp
from jax import lax
from jax.experimental import pallas as pl
from jax.experimental.pallas import tpu as pltpu
from jax.experimental.pallas import tpu_sc as plsc

N = 10000
D = 128
E = 160000

NC = 2
NS = 16
NW = NC * NS
CH = 128
EPW = 5120
E_PAD = NW * EPW
N_PAD = 10112
RPT = N_PAD // NS
TRASH = N


CA = 64
CB = 16
ROWS_A = 2 * CA
ROWS_B = 2 * CB
SPLIT_ROW = NS * ROWS_A


def _agg_body(x_hbm, zeros_hbm, garr_hbm, darr_hbm, out_hbm,
              gidx_all, didx_all, rows0, rows1, agg_sh, sem0, sem1):
    cid = lax.axis_index("c")
    sid = lax.axis_index("s")
    r0 = sid * RPT
    is0 = cid == 0
    base = jnp.where(is0, sid * ROWS_A, SPLIT_ROW + sid * ROWS_B)
    n_chunks = jnp.where(is0, CA, CB)

    pltpu.sync_copy(zeros_hbm.at[pl.ds(r0, RPT)], agg_sh.at[pl.ds(r0, RPT)])
    plsc.subcore_barrier()

    def start(j, buf, sem):
        pltpu.async_copy(x_hbm.at[gidx_all.at[j]], buf, sem)

    def wait(buf, sem):
        pltpu.make_async_copy(x_hbm.at[pl.ds(0, CH)], buf, sem).wait()

    def scatter(j, buf):
        pltpu.sync_copy(buf, agg_sh.at[didx_all.at[j]], add=True)

    for p in range(2):
        poff = base + p * n_chunks

        @pl.when(is0)
        def _():
            pltpu.sync_copy(garr_hbm.at[pl.ds(poff, CA)],
                            gidx_all.at[pl.ds(0, CA)])
            pltpu.sync_copy(darr_hbm.at[pl.ds(poff, CA)],
                            didx_all.at[pl.ds(0, CA)])

        @pl.when(jnp.logical_not(is0))
        def _():
            pltpu.sync_copy(garr_hbm.at[pl.ds(poff, CB)],
                            gidx_all.at[pl.ds(0, CB)])
            pltpu.sync_copy(darr_hbm.at[pl.ds(poff, CB)],
                            didx_all.at[pl.ds(0, CB)])

        start(0, rows0, sem0)
        n_pairs = n_chunks // 2

        def body(k, carry):
            j0 = 2 * k
            j1 = j0 + 1
            wait(rows0, sem0)
            start(j1, rows1, sem1)
            scatter(j0, rows0)
            wait(rows1, sem1)

            @pl.when(k < n_pairs - 1)
            def _():
                start(j1 + 1, rows0, sem0)

            scatter(j1, rows1)
            return carry

        lax.fori_loop(0, n_pairs, body, 0)
    plsc.subcore_barrier()
    pltpu.sync_copy(agg_sh.at[pl.ds(r0, RPT)], out_hbm.at[cid, pl.ds(r0, RPT)])


@functools.cache
def _get_agg_call():
    return functools.partial(
        pl.kernel,
        out_type=jax.ShapeDtypeStruct((NC, N_PAD, D), jnp.float32),
        mesh=plsc.VectorSubcoreMesh(core_axis_name="c", subcore_axis_name="s"),
        scratch_types=[
            pltpu.VMEM((CA, CH), jnp.int32),
            pltpu.VMEM((CA, CH), jnp.int32),
            pltpu.VMEM((CH, D), jnp.float32),
            pltpu.VMEM((CH, D), jnp.float32),
            pltpu.VMEM_SHARED((N_PAD, D), jnp.float32),
            pltpu.SemaphoreType.DMA,
            pltpu.SemaphoreType.DMA,
        ],
    )(_agg_body)


def _agg(x, zeros_rows, garr, darr):
    return _get_agg_call()(x, zeros_rows, garr, darr)


_B = 1000
_G = N // _B


def _mm(a, w):
    return jnp.dot(a.astype(jnp.bfloat16), w.astype(jnp.bfloat16),
                   preferred_element_type=jnp.float32)


def _layer0_body(x_ref, ag_ref, w1, b1, w2, b2, bns, bnb, wih, bih, bhh,
                 o_ref):
    t = x_ref[...] + ag_ref[0] + ag_ref[1]
    h = jnp.maximum(_mm(t, w1[...]) + b1[...], 0.0)
    xc = _mm(h, w2[...]) + b2[...]
    xc = xc * bns[...] + bnb[...]
    gi = _mm(xc, wih[...]) + bih[...]
    gh = bhh[...]
    r = jax.nn.sigmoid(gi[:, :D] + gh[:, :D])
    z = jax.nn.sigmoid(gi[:, D:2 * D] + gh[:, D:2 * D])
    n = jnp.tanh(gi[:, 2 * D:] + r * gh[:, 2 * D:])
    o_ref[...] = (1.0 - z) * n


def _layer1_body(x_ref, ag_ref, w1, b1, w2, b2, bns, bnb, wih, whh, bih, bhh,
                 lw1, lb1, lw2, lb2, o_ref):
    xb = x_ref[...]
    t = xb + ag_ref[0] + ag_ref[1]
    h = jnp.maximum(_mm(t, w1[...]) + b1[...], 0.0)
    xc = _mm(h, w2[...]) + b2[...]
    xc = xc * bns[...] + bnb[...]
    gi = _mm(xc, wih[...]) + bih[...]
    gh = _mm(xb, whh[...]) + bhh[...]
    r = jax.nn.sigmoid(gi[:, :D] + gh[:, :D])
    z = jax.nn.sigmoid(gi[:, D:2 * D] + gh[:, D:2 * D])
    n = jnp.tanh(gi[:, 2 * D:] + r * gh[:, 2 * D:])
    hn = (1.0 - z) * n + z * xb
    u = jnp.maximum(_mm(hn, lw1[...]) + lb1[...], 0.0)
    o_ref[...] = _mm(u, lw2[...]) + lb2[...]


def _w_spec(shape):
    return pl.BlockSpec(shape, lambda i: (0,) * len(shape))


_x_spec = pl.BlockSpec((_B, D), lambda i: (i, 0))
_ag_spec = pl.BlockSpec((NC, _B, D), lambda i: (0, i, 0))

_layer0_call = pl.pallas_call(
    _layer0_body,
    grid=(_G,),
    in_specs=[
        _x_spec, _ag_spec,
        _w_spec((D, D)), _w_spec((1, D)), _w_spec((D, D)), _w_spec((1, D)),
        _w_spec((1, D)), _w_spec((1, D)),
        _w_spec((D, 3 * D)), _w_spec((1, 3 * D)), _w_spec((1, 3 * D)),
    ],
    out_specs=_x_spec,
    out_shape=jax.ShapeDtypeStruct((N, D), jnp.float32),
)

_layer1_call = pl.pallas_call(
    _layer1_body,
    grid=(_G,),
    in_specs=[
        _x_spec, _ag_spec,
        _w_spec((D, D)), _w_spec((1, D)), _w_spec((D, D)), _w_spec((1, D)),
        _w_spec((1, D)), _w_spec((1, D)),
        _w_spec((D, 3 * D)), _w_spec((D, 3 * D)),
        _w_spec((1, 3 * D)), _w_spec((1, 3 * D)),
        _w_spec((D, D)), _w_spec((1, D)), _w_spec((D, D)), _w_spec((1, D)),
    ],
    out_specs=_x_spec,
    out_shape=jax.ShapeDtypeStruct((N, D), jnp.float32),
)


def kernel(x, edge_index, mlp0_W1, mlp0_b1, mlp0_W2, mlp0_b2, bn0_gamma,
           bn0_beta, mlp1_W1, mlp1_b1, mlp1_W2, mlp1_b2, bn1_gamma, bn1_beta,
           gru_W_ih, gru_W_hh, gru_b_ih, gru_b_hh, last_W1, last_b1, last_W2,
           last_b2):
    src = edge_index[0]
    dst = edge_index[1]

    a = jnp.minimum(src, dst)
    b = jnp.maximum(src, dst)
    skey = lax.sort(a * N + b, is_stable=False)
    a_s = skey // N
    b_s = skey - a_s * N
    uniq = jnp.concatenate(
        [jnp.ones((1,), jnp.bool_), skey[1:] != skey[:-1]])
    da = jnp.where(uniq, a_s, TRASH)
    db = jnp.where(uniq & (a_s != b_s), b_s, TRASH)

    pad_i = jnp.zeros((E_PAD - E,), jnp.int32)
    pad_t = jnp.full((E_PAD - E,), TRASH, jnp.int32)
    ga = jnp.concatenate([b_s.astype(jnp.int32), pad_i])
    gb = jnp.concatenate([a_s.astype(jnp.int32), pad_i])
    da = jnp.concatenate([da.astype(jnp.int32), pad_t])
    db = jnp.concatenate([db.astype(jnp.int32), pad_t])
    S = NS * CA * CH
    garr = jnp.concatenate([
        jnp.concatenate([ga[:S].reshape(NS, CA, CH),
                         gb[:S].reshape(NS, CA, CH)], axis=1).reshape(-1, CH),
        jnp.concatenate([ga[S:].reshape(NS, CB, CH),
                         gb[S:].reshape(NS, CB, CH)], axis=1).reshape(-1, CH),
    ])
    darr = jnp.concatenate([
        jnp.concatenate([da[:S].reshape(NS, CA, CH),
                         db[:S].reshape(NS, CA, CH)], axis=1).reshape(-1, CH),
        jnp.concatenate([da[S:].reshape(NS, CB, CH),
                         db[S:].reshape(NS, CB, CH)], axis=1).reshape(-1, CH),
    ])
    zeros_rows = jnp.zeros((N_PAD, D), jnp.float32)

    wihT = gru_W_ih.T
    whhT = gru_W_hh.T
    bih = gru_b_ih.reshape(1, 3 * D)
    bhh = gru_b_hh.reshape(1, 3 * D)
    inv = 1.0 / jnp.sqrt(jnp.float32(1.0 + 1e-5))
    bns0 = (bn0_gamma * inv).reshape(1, D)
    bns1 = (bn1_gamma * inv).reshape(1, D)

    x_r = lax.reduce_precision(x, exponent_bits=8, mantissa_bits=7)
    aggp0 = _agg(x_r, zeros_rows, garr, darr)
    x1 = _layer0_call(
        x, aggp0, mlp0_W1, mlp0_b1.reshape(1, D), mlp0_W2,
        mlp0_b2.reshape(1, D), bns0, bn0_beta.reshape(1, D), wihT, bih, bhh)
    x1_r = lax.reduce_precision(x1, exponent_bits=8, mantissa_bits=7)
    aggp1 = _agg(x1_r, zeros_rows, garr, darr)
    out = _layer1_call(
        x1, aggp1, mlp1_W1, mlp1_b1.reshape(1, D), mlp1_W2,
        mlp1_b2.reshape(1, D), bns1, bn1_beta.reshape(1, D), wihT, whhT, bih,
        bhh, last_W1, last_b1.reshape(1, D), last_W2, last_b2.reshape(1, D))
    return out
